# Initial kernel scaffold; baseline (speedup 1.0000x reference)
import jax
import jax.numpy as jnp
from jax.experimental import pallas as pl


def kernel(x, Wg, w1, w3, w2):
    raise NotImplementedError("write your pallas kernel here")


# speedup vs baseline: 3.9612x; 3.9612x over previous
import functools

import jax
import jax.numpy as jnp
from jax.experimental import pallas as pl
from jax.experimental.pallas import tpu as pltpu

N_EXPERTS = 64
D_MODEL = 768
D_FF = 1024


def _gate_body(x_ref, wg_ref, e0_ref, e1_ref, w0_ref, w1_ref):
    xb = x_ref[...]
    wg = wg_ref[...]
    logits = jax.lax.dot_general(
        xb, wg, (((1,), (1,)), ((), ())), preferred_element_type=jnp.float32)
    mx = jnp.max(logits, axis=1, keepdims=True)
    ex = jnp.exp(logits - mx)
    s = ex / jnp.sum(ex, axis=1, keepdims=True)
    iota = jax.lax.broadcasted_iota(jnp.int32, s.shape, 1)
    m0 = jnp.max(s, axis=1, keepdims=True)
    i0 = jnp.min(jnp.where(s == m0, iota, N_EXPERTS), axis=1, keepdims=True)
    s2 = jnp.where(iota == i0, -jnp.inf, s)
    m1 = jnp.max(s2, axis=1, keepdims=True)
    i1 = jnp.min(jnp.where(s2 == m1, iota, N_EXPERTS), axis=1, keepdims=True)
    tot = m0 + m1
    e0_ref[...] = i0
    e1_ref[...] = i1
    w0_ref[...] = m0 / tot
    w1_ref[...] = m1 / tot


def _moe_body(x_ref, w1_ref, w3_ref, w2_ref, e0_ref, e1_ref, g0_ref, g1_ref,
              out_ref):
    e = pl.program_id(0)
    coef = (jnp.where(e0_ref[...] == e, g0_ref[...], 0.0)
            + jnp.where(e1_ref[...] == e, g1_ref[...], 0.0))
    xb = x_ref[...]
    g = jax.lax.dot_general(
        xb, w1_ref[0], (((1,), (1,)), ((), ())),
        preferred_element_type=jnp.float32)
    u = jax.lax.dot_general(
        xb, w3_ref[0], (((1,), (1,)), ((), ())),
        preferred_element_type=jnp.float32)
    h = (g * jax.nn.sigmoid(g)) * u
    o = jax.lax.dot_general(
        h, w2_ref[0], (((1,), (1,)), ((), ())),
        preferred_element_type=jnp.float32)

    @pl.when(e == 0)
    def _():
        out_ref[...] = jnp.zeros_like(out_ref)

    out_ref[...] += coef * o


@jax.jit
def kernel(x, Wg, w1, w3, w2):
    B, S, D = x.shape
    T = B * S
    x_flat = x.reshape(T, D)

    e0, e1, g0, g1 = pl.pallas_call(
        _gate_body,
        out_shape=(
            jax.ShapeDtypeStruct((T, 1), jnp.int32),
            jax.ShapeDtypeStruct((T, 1), jnp.int32),
            jax.ShapeDtypeStruct((T, 1), jnp.float32),
            jax.ShapeDtypeStruct((T, 1), jnp.float32),
        ),
    )(x_flat, Wg)

    out = pl.pallas_call(
        _moe_body,
        grid=(N_EXPERTS,),
        in_specs=[
            pl.BlockSpec((T, D), lambda e: (0, 0)),
            pl.BlockSpec((1, D_FF, D), lambda e: (e, 0, 0)),
            pl.BlockSpec((1, D_FF, D), lambda e: (e, 0, 0)),
            pl.BlockSpec((1, D, D_FF), lambda e: (e, 0, 0)),
            pl.BlockSpec((T, 1), lambda e: (0, 0)),
            pl.BlockSpec((T, 1), lambda e: (0, 0)),
            pl.BlockSpec((T, 1), lambda e: (0, 0)),
            pl.BlockSpec((T, 1), lambda e: (0, 0)),
        ],
        out_specs=pl.BlockSpec((T, D), lambda e: (0, 0)),
        out_shape=jax.ShapeDtypeStruct((T, D), jnp.float32),
    )(x_flat, w1, w3, w2, e0, e1, g0, g1)

    return out.reshape(B, S, D)


# speedup vs baseline: 11.8512x; 2.9918x over previous
import functools

import jax
import jax.numpy as jnp
from jax import lax
from jax.experimental import pallas as pl
from jax.experimental.pallas import tpu as pltpu
from jax.experimental.pallas import tpu_sc as plsc

N_EXPERTS = 64
D_MODEL = 768
D_FF = 1024
T_TOKENS = 2048
TOP_K = 2
BT = 128
NW = 32
CT = T_TOKENS // NW
P_ROWS = T_TOKENS * TOP_K
P_PAD = P_ROWS + N_EXPERTS * 8 + BT
LC = 16


def _gate_body(x_ref, wg_ref, e0_ref, e1_ref, w0_ref, w1_ref,
               hists_ref, offs_ref):
    xb = x_ref[...]
    wg = wg_ref[...]
    logits = lax.dot_general(
        xb, wg, (((1,), (1,)), ((), ())), preferred_element_type=jnp.float32)
    mx = jnp.max(logits, axis=1, keepdims=True)
    ex = jnp.exp(logits - mx)
    s = ex / jnp.sum(ex, axis=1, keepdims=True)
    iota = lax.broadcasted_iota(jnp.int32, s.shape, 1)
    m0 = jnp.max(s, axis=1, keepdims=True)
    i0 = jnp.min(jnp.where(s == m0, iota, N_EXPERTS), axis=1, keepdims=True)
    s2 = jnp.where(iota == i0, -jnp.inf, s)
    m1 = jnp.max(s2, axis=1, keepdims=True)
    i1 = jnp.min(jnp.where(s2 == m1, iota, N_EXPERTS), axis=1, keepdims=True)
    tot = m0 + m1
    e0_ref[...] = i0
    e1_ref[...] = i1
    w0_ref[...] = m0 / tot
    w1_ref[...] = m1 / tot

    one = jnp.float32(1.0)
    zerof = jnp.float32(0.0)
    oh = (jnp.where(iota == i0, one, zerof)
          + jnp.where(iota == i1, one, zerof))
    tsel_r = lax.broadcasted_iota(jnp.int32, (NW, T_TOKENS), 0)
    tsel_c = lax.broadcasted_iota(jnp.int32, (NW, T_TOKENS), 1)
    sel = jnp.where(tsel_c // CT == tsel_r, one, zerof)
    hists_f = lax.dot_general(sel, oh, (((1,), (0,)), ((), ())),
                              preferred_element_type=jnp.float32)
    hists_ref[...] = hists_f.astype(jnp.int32)
    htot = jnp.sum(hists_f, axis=0, keepdims=True)
    htotp = jnp.bitwise_and(htot.astype(jnp.int32) + 7, jnp.int32(-8))
    htotp_f = htotp.astype(jnp.float32)
    er = lax.broadcasted_iota(jnp.int32, (N_EXPERTS, N_EXPERTS), 0)
    ec = lax.broadcasted_iota(jnp.int32, (N_EXPERTS, N_EXPERTS), 1)
    strict_lt = jnp.where(er < ec, one, zerof)
    excl = lax.dot_general(htotp_f, strict_lt, (((1,), (0,)), ((), ())),
                           preferred_element_type=jnp.float32)
    total = jnp.sum(htotp_f, axis=1, keepdims=True)
    pad = jnp.zeros((1, 15), jnp.float32)
    offs_ref[...] = jnp.concatenate([excl, total, pad], axis=1).astype(jnp.int32)


def _dup_stats(ev, lane):
    zero = jnp.zeros((LC,), jnp.int32)
    one = zero + 1
    rank = zero
    total = zero
    for j in range(LC):
        ej = ev[j]
        m = jnp.where(ev == ej, one, zero)
        rank = rank + jnp.where(lane > j, m, zero)
        total = total + m
    return rank, total


def _route_body(e0_hbm, e1_hbm, x_hbm, hists_hbm, offs_hbm,
                xs_hbm, ie_hbm, io_hbm,
                ids0_vm, ids1_vm, allh_vm, base_vm, off_vm,
                ie_vm, io_vm, xrows_vm, sem, semx):
    wid = lax.axis_index("s") * 2 + lax.axis_index("c")
    t0 = wid * CT
    lane = lax.broadcasted_iota(jnp.int32, (LC,), 0)
    zero = jnp.zeros((LC,), jnp.int32)

    xcopy = pltpu.async_copy(x_hbm.at[pl.ds(t0, CT)], xrows_vm, semx)

    pltpu.sync_copy(e0_hbm.at[pl.ds(t0, CT)], ids0_vm)
    pltpu.sync_copy(e1_hbm.at[pl.ds(t0, CT)], ids1_vm)
    pltpu.sync_copy(hists_hbm, allh_vm)
    pltpu.sync_copy(offs_hbm, off_vm)

    widv = zero + wid
    for c in range(N_EXPERTS // LC):
        sl = pl.ds(c * LC, LC)
        pre = zero
        for t in range(NW):
            v = allh_vm[t, sl]
            tv = jnp.full((LC,), t, jnp.int32)
            pre = pre + jnp.where(tv < widv, v, zero)
        base_vm[sl] = off_vm[sl] + pre

    for src, dst in ((ids0_vm, ie_vm), (ids1_vm, io_vm)):
        for c in range(CT // LC):
            sl = pl.ds(c * LC, LC)
            ev = src[sl]
            rank, total = _dup_stats(ev, lane)
            b = plsc.load_gather(base_vm, [ev])
            dst[sl] = b + rank
            plsc.store_scatter(base_vm, [ev], b + total)

    pltpu.sync_copy(ie_vm, ie_hbm.at[pl.ds(t0, CT)])
    pltpu.sync_copy(io_vm, io_hbm.at[pl.ds(t0, CT)])

    xcopy.wait()
    pltpu.async_copy(xrows_vm, xs_hbm.at[ie_vm], sem).wait()
    pltpu.async_copy(xrows_vm, xs_hbm.at[io_vm], sem).wait()


def _gemm_body(offs_ref, xs_ref, w1_ref, w3_ref, w2_ref, out_ref):
    e = pl.program_id(0)
    start = offs_ref[e]
    end = offs_ref[e + 1]
    nb = (end - start + BT - 1) // BT

    def body(j, carry):
        rs = pl.multiple_of(start + j * BT, 8)
        xb = xs_ref[pl.ds(rs, BT), :]
        g = lax.dot_general(xb, w1_ref[0], (((1,), (1,)), ((), ())),
                            preferred_element_type=jnp.float32)
        u = lax.dot_general(xb, w3_ref[0], (((1,), (1,)), ((), ())),
                            preferred_element_type=jnp.float32)
        h = (g * jax.nn.sigmoid(g)) * u
        o = lax.dot_general(h, w2_ref[0], (((1,), (1,)), ((), ())),
                            preferred_element_type=jnp.float32)
        out_ref[pl.ds(rs, BT), :] = o
        return carry

    lax.fori_loop(0, nb, body, 0)


def _combine_body(os_hbm, ie_hbm, io_hbm, w0_hbm, w1_hbm,
                  out_hbm,
                  ie_vm, io_vm, w0_vm, w1_vm, rowsa_vm, rowsb_vm, sema, semb):
    wid = lax.axis_index("s") * 2 + lax.axis_index("c")
    t0 = wid * CT

    pltpu.sync_copy(ie_hbm.at[pl.ds(t0, CT)], ie_vm)
    pltpu.sync_copy(io_hbm.at[pl.ds(t0, CT)], io_vm)
    pltpu.sync_copy(w0_hbm.at[pl.ds(t0, CT)], w0_vm)
    pltpu.sync_copy(w1_hbm.at[pl.ds(t0, CT)], w1_vm)

    ca = pltpu.async_copy(os_hbm.at[ie_vm], rowsa_vm, sema)
    cb = pltpu.async_copy(os_hbm.at[io_vm], rowsb_vm, semb)
    ca.wait()

    def scale_a(tt, carry):
        tv = jnp.zeros((LC,), jnp.int32) + tt
        wv0 = plsc.load_gather(w0_vm, [tv])
        for c in range(D_MODEL // LC):
            sl = pl.ds(c * LC, LC)
            rowsa_vm[tt, sl] = rowsa_vm[tt, sl] * wv0
        return carry

    lax.fori_loop(0, CT, scale_a, 0)
    cb.wait()

    def add_b(tt, carry):
        tv = jnp.zeros((LC,), jnp.int32) + tt
        wv1 = plsc.load_gather(w1_vm, [tv])
        for c in range(D_MODEL // LC):
            sl = pl.ds(c * LC, LC)
            rowsa_vm[tt, sl] = rowsa_vm[tt, sl] + rowsb_vm[tt, sl] * wv1
        return carry

    lax.fori_loop(0, CT, add_b, 0)

    pltpu.sync_copy(rowsa_vm, out_hbm.at[pl.ds(t0, CT)])


_SC_MESH = plsc.VectorSubcoreMesh(core_axis_name="c", subcore_axis_name="s")

_route_call = functools.partial(
    pl.kernel, mesh=_SC_MESH,
    compiler_params=pltpu.CompilerParams(needs_layout_passes=False),
    out_type=(
        jax.ShapeDtypeStruct((P_PAD, D_MODEL), jnp.float32),
        jax.ShapeDtypeStruct((T_TOKENS,), jnp.int32),
        jax.ShapeDtypeStruct((T_TOKENS,), jnp.int32),
    ),
    scratch_types=[
        pltpu.VMEM((CT,), jnp.int32),
        pltpu.VMEM((CT,), jnp.int32),
        pltpu.VMEM((NW, N_EXPERTS), jnp.int32),
        pltpu.VMEM((N_EXPERTS,), jnp.int32),
        pltpu.VMEM((80,), jnp.int32),
        pltpu.VMEM((CT,), jnp.int32),
        pltpu.VMEM((CT,), jnp.int32),
        pltpu.VMEM((CT, D_MODEL), jnp.float32),
        pltpu.SemaphoreType.DMA,
        pltpu.SemaphoreType.DMA,
    ],
)(_route_body)

_combine_call = functools.partial(
    pl.kernel, mesh=_SC_MESH,
    compiler_params=pltpu.CompilerParams(needs_layout_passes=False),
    out_type=jax.ShapeDtypeStruct((T_TOKENS, D_MODEL), jnp.float32),
    scratch_types=[
        pltpu.VMEM((CT,), jnp.int32),
        pltpu.VMEM((CT,), jnp.int32),
        pltpu.VMEM((CT,), jnp.float32),
        pltpu.VMEM((CT,), jnp.float32),
        pltpu.VMEM((CT, D_MODEL), jnp.float32),
        pltpu.VMEM((CT, D_MODEL), jnp.float32),
        pltpu.SemaphoreType.DMA,
        pltpu.SemaphoreType.DMA,
    ],
)(_combine_body)


@jax.jit
def kernel(x, Wg, w1, w3, w2):
    B, S, D = x.shape
    T = B * S
    x_flat = x.reshape(T, D)

    e0, e1, g0, g1, hists, offs2d = pl.pallas_call(
        _gate_body,
        out_shape=(
            jax.ShapeDtypeStruct((T, 1), jnp.int32),
            jax.ShapeDtypeStruct((T, 1), jnp.int32),
            jax.ShapeDtypeStruct((T, 1), jnp.float32),
            jax.ShapeDtypeStruct((T, 1), jnp.float32),
            jax.ShapeDtypeStruct((NW, N_EXPERTS), jnp.int32),
            jax.ShapeDtypeStruct((1, 80), jnp.int32),
        ),
    )(x_flat, Wg)

    e0f = e0.reshape(T)
    e1f = e1.reshape(T)
    offs = offs2d.reshape(80)
    xs, ie, io = _route_call(e0f, e1f, x_flat, hists, offs)

    os_sorted = pl.pallas_call(
        _gemm_body,
        grid_spec=pltpu.PrefetchScalarGridSpec(
            num_scalar_prefetch=1,
            grid=(N_EXPERTS,),
            in_specs=[
                pl.BlockSpec((P_PAD, D_MODEL), lambda e, offs: (0, 0)),
                pl.BlockSpec((1, D_FF, D_MODEL), lambda e, offs: (e, 0, 0)),
                pl.BlockSpec((1, D_FF, D_MODEL), lambda e, offs: (e, 0, 0)),
                pl.BlockSpec((1, D_MODEL, D_FF), lambda e, offs: (e, 0, 0)),
            ],
            out_specs=pl.BlockSpec((P_PAD, D_MODEL), lambda e, offs: (0, 0)),
        ),
        out_shape=jax.ShapeDtypeStruct((P_PAD, D_MODEL), jnp.float32),
    )(offs, xs, w1, w3, w2)

    out = _combine_call(os_sorted, ie, io, g0.reshape(T), g1.reshape(T))
    return out.reshape(B, S, D)
